# baseline (speedup 1.0000x reference)
import functools

import jax
import jax.numpy as jnp
from jax import lax
from jax.experimental import pallas as pl
from jax.experimental.pallas import tpu as pltpu
from jax.experimental.pallas import tpu_sc as plsc

DIM = 4096
INTER = 14336
SPARSITY = 256
RANK = 16
NBLK = INTER // SPARSITY
L = 16
NSUB = 16
BPW = 4

_mesh = plsc.VectorSubcoreMesh(core_axis_name="c", subcore_axis_name="s",
                               num_cores=1)


def _butterfly(vec, iota, op):
    for sh in (8, 4, 2, 1):
        perm = jnp.bitwise_xor(iota, sh)
        vec = op(vec, vec[perm])
    return vec


@functools.partial(
    pl.kernel,
    out_type=(
        jax.ShapeDtypeStruct((NBLK,), jnp.int32),
        jax.ShapeDtypeStruct((NSUB, L), jnp.float32),
        jax.ShapeDtypeStruct((NSUB, L), jnp.int32),
    ),
    mesh=_mesh,
    compiler_params=pltpu.CompilerParams(needs_layout_passes=False),
    scratch_types=[
        pltpu.VMEM((DIM,), jnp.float32),
        pltpu.VMEM((DIM,), jnp.float32),
        pltpu.VMEM((BPW * SPARSITY * RANK,), jnp.float32),
        pltpu.VMEM((RANK, L), jnp.float32),
        pltpu.VMEM((L,), jnp.float32),
        pltpu.VMEM((L,), jnp.int32),
        pltpu.VMEM((NSUB, L), jnp.int32),
        pltpu.VMEM((4 * L,), jnp.int32),
        pltpu.SemaphoreType.DMA,
    ],
)
def _sc_controller(x_hbm, w1_hbm, w2_hbm, out_hbm, hstage_hbm, rstage_hbm,
                   x_v, w1_v, w2_v, h_v, hst_v, res_v, sres_v, asm_v, sem):
    sid = lax.axis_index("s")
    iota = lax.iota(jnp.int32, L)

    blk0 = jnp.minimum(BPW * sid, NBLK - BPW)

    w2_copy = pltpu.async_copy(
        w2_hbm.at[pl.ds(blk0 * (SPARSITY * RANK), BPW * SPARSITY * RANK)],
        w2_v, sem)
    pltpu.sync_copy(x_hbm.at[0], x_v)
    pltpu.sync_copy(w1_hbm.at[sid], w1_v)

    def p1_body(j, acc):
        o = j * L
        return acc + x_v[pl.ds(o, L)] * w1_v[pl.ds(o, L)]

    acc = lax.fori_loop(0, DIM // L, p1_body,
                        jnp.zeros((L,), jnp.float32), unroll=8)
    hst_v[...] = _butterfly(acc, iota, jnp.add)
    pltpu.sync_copy(hst_v, hstage_hbm.at[sid])
    plsc.subcore_barrier()

    pltpu.sync_copy(hstage_hbm, h_v)
    hs = [h_v[r] for r in range(RANK)]

    w2_copy.wait()

    answers = []
    for blk in range(BPW):
        base = blk * SPARSITY

        def g_body(g, carry, base=base):
            bv, bi = carry
            fidx = (base + g * L) * RANK + iota * RANK
            vals = jnp.zeros((L,), jnp.float32)
            for r in range(RANK):
                col = plsc.load_gather(w2_v, [fidx + r])
                vals = vals + hs[r] * col
            lidx = g * L + iota
            pred = vals > bv
            return jnp.where(pred, vals, bv), jnp.where(pred, lidx, bi)

        bv, bi = lax.fori_loop(
            0, SPARSITY // L, g_body,
            (jnp.full((L,), -jnp.inf, jnp.float32),
             jnp.zeros((L,), jnp.int32)))
        m = _butterfly(bv, iota, jnp.maximum)
        cand = jnp.where(bv == m, bi, jnp.int32(1 << 30))
        loc = _butterfly(cand, iota, jnp.minimum)
        answers.append(loc + (blk0 + blk) * SPARSITY)

    res = jnp.where(iota == 3, answers[3], answers[2])
    res = jnp.where(iota == 1, answers[1], res)
    res_v[...] = jnp.where(iota == 0, answers[0], res)
    pltpu.sync_copy(res_v, rstage_hbm.at[sid])
    plsc.subcore_barrier()

    quarter = iota >> 2
    par = jnp.bitwise_and(iota, 3)

    @pl.when(sid == 0)
    def _():
        pltpu.sync_copy(rstage_hbm, sres_v)
        for q in range(4):
            asm_v[pl.ds(q * L, L)] = plsc.load_gather(
                sres_v, [4 * q + quarter, par])
        pltpu.sync_copy(asm_v.at[pl.ds(0, NBLK)], out_hbm)


def kernel(x, W1, W2):
    return _sc_controller(x, W1, W2.reshape(-1))[0]


# speedup vs baseline: 1.0287x; 1.0287x over previous
import functools

import jax
import jax.numpy as jnp
from jax import lax
from jax.experimental import pallas as pl
from jax.experimental.pallas import tpu as pltpu
from jax.experimental.pallas import tpu_sc as plsc

DIM = 4096
INTER = 14336
SPARSITY = 256
RANK = 16
NBLK = INTER // SPARSITY
L = 16
NSUB = 16
BPW = 4

_mesh = plsc.VectorSubcoreMesh(core_axis_name="c", subcore_axis_name="s",
                               num_cores=1)


def _butterfly(vec, iota, op):
    for sh in (8, 4, 2, 1):
        perm = jnp.bitwise_xor(iota, sh)
        vec = op(vec, vec[perm])
    return vec


@functools.partial(
    pl.kernel,
    out_type=(
        jax.ShapeDtypeStruct((NBLK,), jnp.int32),
        jax.ShapeDtypeStruct((NSUB, L), jnp.float32),
        jax.ShapeDtypeStruct((NSUB, L), jnp.int32),
    ),
    mesh=_mesh,
    compiler_params=pltpu.CompilerParams(needs_layout_passes=False),
    scratch_types=[
        pltpu.VMEM((DIM,), jnp.float32),
        pltpu.VMEM((DIM,), jnp.float32),
        pltpu.VMEM((BPW * SPARSITY * RANK,), jnp.float32),
        pltpu.VMEM((RANK, L), jnp.float32),
        pltpu.VMEM((L,), jnp.float32),
        pltpu.VMEM((L,), jnp.int32),
        pltpu.VMEM((NSUB, L), jnp.int32),
        pltpu.VMEM((4 * L,), jnp.int32),
        pltpu.SemaphoreType.DMA,
        pltpu.SemaphoreType.DMA,
        pltpu.SemaphoreType.DMA,
    ],
)
def _sc_controller(x_hbm, w1_hbm, w2_hbm, out_hbm, hstage_hbm, rstage_hbm,
                   x_v, w1_v, w2_v, h_v, hst_v, res_v, sres_v, asm_v,
                   sem, xsem, w1sem):
    sid = lax.axis_index("s")
    iota = lax.iota(jnp.int32, L)

    blk0 = jnp.minimum(BPW * sid, NBLK - BPW)

    w2_copy = pltpu.async_copy(
        w2_hbm.at[pl.ds(blk0 * (SPARSITY * RANK), BPW * SPARSITY * RANK)],
        w2_v, sem)
    x_copy = pltpu.async_copy(x_hbm.at[0], x_v, xsem)
    w1_copy = pltpu.async_copy(w1_hbm.at[sid], w1_v, w1sem)
    x_copy.wait()
    w1_copy.wait()

    def p1_body(j, acc):
        o = j * L
        return acc + x_v[pl.ds(o, L)] * w1_v[pl.ds(o, L)]

    acc = lax.fori_loop(0, DIM // L, p1_body,
                        jnp.zeros((L,), jnp.float32), unroll=16)
    hst_v[...] = _butterfly(acc, iota, jnp.add)
    pltpu.sync_copy(hst_v, hstage_hbm.at[sid])
    plsc.subcore_barrier()

    pltpu.sync_copy(hstage_hbm, h_v)
    hs = [h_v[r] for r in range(RANK)]

    w2_copy.wait()

    answers = []
    for blk in range(BPW):
        base = blk * SPARSITY

        def g_body(g, carry, base=base):
            bv, bi = carry
            fidx = (base + g * L) * RANK + iota * RANK
            vals = jnp.zeros((L,), jnp.float32)
            for r in range(RANK):
                col = plsc.load_gather(w2_v, [fidx + r])
                vals = vals + hs[r] * col
            lidx = g * L + iota
            pred = vals > bv
            return jnp.where(pred, vals, bv), jnp.where(pred, lidx, bi)

        bv, bi = lax.fori_loop(
            0, SPARSITY // L, g_body,
            (jnp.full((L,), -jnp.inf, jnp.float32),
             jnp.zeros((L,), jnp.int32)))
        m = _butterfly(bv, iota, jnp.maximum)
        cand = jnp.where(bv == m, bi, jnp.int32(1 << 30))
        loc = _butterfly(cand, iota, jnp.minimum)
        answers.append(loc + (blk0 + blk) * SPARSITY)

    res = jnp.where(iota == 3, answers[3], answers[2])
    res = jnp.where(iota == 1, answers[1], res)
    res_v[...] = jnp.where(iota == 0, answers[0], res)
    pltpu.sync_copy(res_v, rstage_hbm.at[sid])
    plsc.subcore_barrier()

    quarter = iota >> 2
    par = jnp.bitwise_and(iota, 3)

    @pl.when(sid == 0)
    def _():
        pltpu.sync_copy(rstage_hbm, sres_v)
        for q in range(4):
            asm_v[pl.ds(q * L, L)] = plsc.load_gather(
                sres_v, [4 * q + quarter, par])
        pltpu.sync_copy(asm_v.at[pl.ds(0, NBLK)], out_hbm)


def kernel(x, W1, W2):
    return _sc_controller(x, W1, W2.reshape(-1))[0]
